# baseline (speedup 1.0000x reference)
import functools

import jax
import jax.numpy as jnp
from jax import lax
from jax.experimental import pallas as pl
from jax.experimental.pallas import tpu as pltpu
from jax.experimental.pallas import tpu_sc as plsc

_GROUP = 128
_JS_PER_CHUNK = 10
_EMBP = 64


@functools.lru_cache(maxsize=None)
def _make_gather(batch: int, enc: int, vocab: int):
    info = plsc.get_sparse_core_info()
    nw = info.num_cores * info.num_subcores
    assert batch % (nw * _GROUP) == 0 and enc % _JS_PER_CHUNK == 0
    chunks = enc // _JS_PER_CHUNK
    chunk_rows = _GROUP * _JS_PER_CHUNK

    mesh = plsc.VectorSubcoreMesh(core_axis_name="c", subcore_axis_name="s")

    @functools.partial(
        pl.kernel,
        mesh=mesh,
        out_type=jax.ShapeDtypeStruct((batch * enc // 2, 2 * _EMBP), jnp.bfloat16),
        scratch_types=[
            pltpu.VMEM((_JS_PER_CHUNK, _GROUP), jnp.int32),
            pltpu.VMEM((chunk_rows, _EMBP), jnp.bfloat16),
            pltpu.SemaphoreType.DMA,
            pltpu.SemaphoreType.DMA,
        ],
        compiler_params=pltpu.CompilerParams(use_tc_tiling_on_sc=False),
    )
    def gather_k(table_hbm, idx_hbm, out_hbm, idx_v, rows_v, gsem, osem):
        cid = lax.axis_index("c")
        sid = lax.axis_index("s")
        wid = sid * info.num_cores + cid
        b0 = wid * _GROUP

        def chunk_body(c, carry):
            j0 = c * _JS_PER_CHUNK
            pltpu.sync_copy(
                idx_hbm.at[pl.ds(j0, _JS_PER_CHUNK), pl.ds(b0, _GROUP)], idx_v
            )
            handles = []
            for j in range(_JS_PER_CHUNK):
                handles.append(
                    pltpu.async_copy(
                        table_hbm.at[idx_v.at[j]],
                        rows_v.at[pl.ds(j * _GROUP, _GROUP)],
                        gsem,
                    )
                )
            for h in handles:
                h.wait()
            writes = []
            for j in range(_JS_PER_CHUNK):
                p = (j0 + j) // 2
                h = (j0 + j) % 2
                writes.append(
                    pltpu.make_async_copy(
                        rows_v.at[pl.ds(j * _GROUP, _GROUP)],
                        out_hbm.at[
                            pl.ds(batch * p + b0, _GROUP),
                            pl.ds(h * _EMBP, _EMBP),
                        ],
                        osem,
                    )
                )
            for wcp in writes:
                wcp.start()
            for wcp in writes:
                wcp.wait()
            return carry

        lax.fori_loop(0, chunks, chunk_body, 0)

    return gather_k


def _matmul_kernel(a_ref, w_ref, b_ref, o_ref):
    p = pl.program_id(1)
    acc = lax.dot_general(
        a_ref[...], w_ref[...],
        dimension_numbers=(((1,), (1,)), ((), ())),
        preferred_element_type=jnp.float32,
    )

    @pl.when(p == 0)
    def _():
        o_ref[...] = jnp.broadcast_to(b_ref[...], o_ref.shape)

    o_ref[...] += acc


def _tc_matmul(gathered, Wp, b2, batch):
    out_dim = Wp.shape[0]
    npairs = Wp.shape[1] // (2 * _EMBP)
    bm = 512
    nblocks = batch // bm
    return pl.pallas_call(
        _matmul_kernel,
        grid=(nblocks, npairs),
        in_specs=[
            pl.BlockSpec((bm, 2 * _EMBP), lambda i, p: (p * nblocks + i, 0)),
            pl.BlockSpec((out_dim, 2 * _EMBP), lambda i, p: (0, p)),
            pl.BlockSpec((1, out_dim), lambda i, p: (0, 0)),
        ],
        out_specs=pl.BlockSpec((bm, out_dim), lambda i, p: (i, 0)),
        out_shape=jax.ShapeDtypeStruct((batch, out_dim), jnp.float32),
    )(gathered, Wp, b2)


def kernel(x, table, W, b):
    batch, enc = x.shape
    vocab, emb = table.shape
    out_dim = W.shape[0]

    idx = x.T.astype(jnp.int32)

    table_p = jnp.pad(
        table.astype(jnp.bfloat16), ((0, 0), (0, _EMBP - emb))
    ).reshape(vocab // 2, 2 * _EMBP)
    gathered = _make_gather(batch, enc, vocab)(
        table_p.reshape(vocab, _EMBP), idx
    )

    Wp = jnp.pad(
        W.astype(jnp.bfloat16).reshape(out_dim, enc, emb),
        ((0, 0), (0, 0), (0, _EMBP - emb)),
    ).reshape(out_dim, enc * _EMBP)

    return _tc_matmul(gathered, Wp, b.reshape(1, out_dim), batch)


# speedup vs baseline: 1.6151x; 1.6151x over previous
import functools

import jax
import jax.numpy as jnp
from jax import lax
from jax.experimental import pallas as pl
from jax.experimental.pallas import tpu as pltpu
from jax.experimental.pallas import tpu_sc as plsc

_GROUP = 128
_JS_PER_CHUNK = 10
_EMBP = 64


@functools.lru_cache(maxsize=None)
def _make_gather(batch: int, enc: int, vocab: int):
    info = plsc.get_sparse_core_info()
    nw = info.num_cores * info.num_subcores
    assert batch % (nw * _GROUP) == 0 and enc % _JS_PER_CHUNK == 0
    chunks = enc // _JS_PER_CHUNK
    chunk_rows = _GROUP * _JS_PER_CHUNK

    mesh = plsc.VectorSubcoreMesh(core_axis_name="c", subcore_axis_name="s")

    @functools.partial(
        pl.kernel,
        mesh=mesh,
        out_type=jax.ShapeDtypeStruct((batch * enc // 2, 2 * _EMBP), jnp.float32),
        scratch_types=[
            pltpu.VMEM((_JS_PER_CHUNK, _GROUP), jnp.int32),
            pltpu.VMEM((chunk_rows, _EMBP), jnp.float32),
            pltpu.SemaphoreType.DMA,
            pltpu.SemaphoreType.DMA,
        ],
        compiler_params=pltpu.CompilerParams(use_tc_tiling_on_sc=False),
    )
    def gather_k(table_hbm, idx_hbm, out_hbm, idx_v, rows_v, gsem, osem):
        cid = lax.axis_index("c")
        sid = lax.axis_index("s")
        wid = sid * info.num_cores + cid
        b0 = wid * _GROUP

        def chunk_body(c, carry):
            j0 = c * _JS_PER_CHUNK
            pltpu.sync_copy(
                idx_hbm.at[pl.ds(j0, _JS_PER_CHUNK), pl.ds(b0, _GROUP)], idx_v
            )
            handles = []
            for j in range(_JS_PER_CHUNK):
                handles.append(
                    pltpu.async_copy(
                        table_hbm.at[idx_v.at[j]],
                        rows_v.at[pl.ds(j * _GROUP, _GROUP)],
                        gsem,
                    )
                )
            for h in handles:
                h.wait()
            writes = []
            for j in range(_JS_PER_CHUNK):
                p = (j0 + j) // 2
                h = (j0 + j) % 2
                writes.append(
                    pltpu.make_async_copy(
                        rows_v.at[pl.ds(j * _GROUP, _GROUP)],
                        out_hbm.at[
                            pl.ds(batch * p + b0, _GROUP),
                            pl.ds(h * _EMBP, _EMBP),
                        ],
                        osem,
                    )
                )
            for wcp in writes:
                wcp.start()
            for wcp in writes:
                wcp.wait()
            return carry

        lax.fori_loop(0, chunks, chunk_body, 0)

    return gather_k


def _matmul_kernel(a_ref, w_ref, b_ref, o_ref):
    p = pl.program_id(1)
    acc = lax.dot_general(
        a_ref[...].astype(jnp.bfloat16), w_ref[...].astype(jnp.bfloat16),
        dimension_numbers=(((1,), (1,)), ((), ())),
        preferred_element_type=jnp.float32,
    )

    @pl.when(p == 0)
    def _():
        o_ref[...] = jnp.broadcast_to(b_ref[...], o_ref.shape)

    o_ref[...] += acc


def _tc_matmul(gathered, Wp, b2, batch):
    out_dim = Wp.shape[0]
    npairs = Wp.shape[1] // (2 * _EMBP)
    bm = 1024
    nblocks = batch // bm
    return pl.pallas_call(
        _matmul_kernel,
        grid=(nblocks, npairs),
        in_specs=[
            pl.BlockSpec((bm, 2 * _EMBP), lambda i, p: (p * nblocks + i, 0)),
            pl.BlockSpec((out_dim, 2 * _EMBP), lambda i, p: (0, p)),
            pl.BlockSpec((1, out_dim), lambda i, p: (0, 0)),
        ],
        out_specs=pl.BlockSpec((bm, out_dim), lambda i, p: (i, 0)),
        out_shape=jax.ShapeDtypeStruct((batch, out_dim), jnp.float32),
    )(gathered, Wp, b2)


def kernel(x, table, W, b):
    batch, enc = x.shape
    vocab, emb = table.shape
    out_dim = W.shape[0]

    idx = x.T.astype(jnp.int32)

    table_p = jnp.pad(table, ((0, 0), (0, _EMBP - emb))).reshape(
        vocab // 2, 2 * _EMBP
    )
    gathered = _make_gather(batch, enc, vocab)(
        table_p.reshape(vocab, _EMBP), idx
    )

    Wp = jnp.pad(
        W.reshape(out_dim, enc, emb), ((0, 0), (0, 0), (0, _EMBP - emb))
    ).reshape(out_dim, enc * _EMBP)

    return _tc_matmul(gathered, Wp, b.reshape(1, out_dim), batch)


# speedup vs baseline: 1.9368x; 1.1992x over previous
import functools

import jax
import jax.numpy as jnp
from jax import lax
from jax.experimental import pallas as pl
from jax.experimental.pallas import tpu as pltpu
from jax.experimental.pallas import tpu_sc as plsc

_GROUP = 128
_JS_PER_CHUNK = 10
_EMBP = 64


@functools.lru_cache(maxsize=None)
def _make_gather(batch: int, enc: int, vocab: int):
    info = plsc.get_sparse_core_info()
    nw = info.num_cores * info.num_subcores
    assert batch % (nw * _GROUP) == 0 and enc % _JS_PER_CHUNK == 0
    chunks = enc // _JS_PER_CHUNK
    chunk_rows = _GROUP * _JS_PER_CHUNK

    mesh = plsc.VectorSubcoreMesh(core_axis_name="c", subcore_axis_name="s")

    @functools.partial(
        pl.kernel,
        mesh=mesh,
        out_type=jax.ShapeDtypeStruct((batch * enc // 2, 2 * _EMBP), jnp.float32),
        scratch_types=[
            pltpu.VMEM((_JS_PER_CHUNK, _GROUP), jnp.int32),
            pltpu.VMEM((chunk_rows, _EMBP), jnp.float32),
            pltpu.SemaphoreType.DMA,
            pltpu.SemaphoreType.DMA,
        ],
        compiler_params=pltpu.CompilerParams(use_tc_tiling_on_sc=False),
    )
    def gather_k(table_hbm, idx_hbm, out_hbm, idx_v, rows_v, gsem, osem):
        cid = lax.axis_index("c")
        sid = lax.axis_index("s")
        wid = sid * info.num_cores + cid
        b0 = wid * _GROUP

        def chunk_body(c, carry):
            j0 = c * _JS_PER_CHUNK
            pltpu.sync_copy(
                idx_hbm.at[pl.ds(j0, _JS_PER_CHUNK), pl.ds(b0, _GROUP)], idx_v
            )
            handles = []
            for j in range(_JS_PER_CHUNK):
                handles.append(
                    pltpu.async_copy(
                        table_hbm.at[idx_v.at[j]],
                        rows_v.at[pl.ds(j * _GROUP, _GROUP)],
                        gsem,
                    )
                )
            for h in handles:
                h.wait()
            writes = []
            for j in range(_JS_PER_CHUNK):
                p = (j0 + j) // 2
                h = (j0 + j) % 2
                writes.append(
                    pltpu.make_async_copy(
                        rows_v.at[pl.ds(j * _GROUP, _GROUP)],
                        out_hbm.at[
                            pl.ds(batch * p + b0, _GROUP),
                            pl.ds(h * _EMBP, _EMBP),
                        ],
                        osem,
                    )
                )
            for wcp in writes:
                wcp.start()
            for wcp in writes:
                wcp.wait()
            return carry

        lax.fori_loop(0, chunks, chunk_body, 0)

    return gather_k


def _pad_kernel(a1_ref, a2_ref, o_ref):
    bv = a1_ref.shape[0]
    emb = a1_ref.shape[1]
    z = jnp.zeros((bv, _EMBP - emb), jnp.float32)
    o_ref[...] = jnp.concatenate([a1_ref[...], z, a2_ref[...], z], axis=1)


def _tc_pad_table(table):
    vocab, emb = table.shape
    bv = 2000
    half_blocks = (vocab // 2) // bv
    return pl.pallas_call(
        _pad_kernel,
        grid=(half_blocks,),
        in_specs=[
            pl.BlockSpec((bv, emb), lambda c: (c, 0)),
            pl.BlockSpec((bv, emb), lambda c, hb=half_blocks: (c + hb, 0)),
        ],
        out_specs=pl.BlockSpec((bv, 2 * _EMBP), lambda c: (c, 0)),
        out_shape=jax.ShapeDtypeStruct((vocab // 2, 2 * _EMBP), jnp.float32),
    )(table, table)


def _matmul_kernel(a_ref, w_ref, b_ref, o_ref):
    p = pl.program_id(1)
    acc = lax.dot_general(
        a_ref[...].astype(jnp.bfloat16), w_ref[...].astype(jnp.bfloat16),
        dimension_numbers=(((1,), (1,)), ((), ())),
        preferred_element_type=jnp.float32,
    )

    @pl.when(p == 0)
    def _():
        o_ref[...] = jnp.broadcast_to(b_ref[...], o_ref.shape)

    o_ref[...] += acc


def _tc_matmul(gathered, Wp, b2, batch):
    out_dim = Wp.shape[0]
    npairs = Wp.shape[1] // (2 * _EMBP)
    bm = 1024
    nblocks = batch // bm
    return pl.pallas_call(
        _matmul_kernel,
        grid=(nblocks, npairs),
        in_specs=[
            pl.BlockSpec((bm, 2 * _EMBP), lambda i, p: (p * nblocks + i, 0)),
            pl.BlockSpec((out_dim, 2 * _EMBP), lambda i, p: (0, p)),
            pl.BlockSpec((1, out_dim), lambda i, p: (0, 0)),
        ],
        out_specs=pl.BlockSpec((bm, out_dim), lambda i, p: (i, 0)),
        out_shape=jax.ShapeDtypeStruct((batch, out_dim), jnp.float32),
    )(gathered, Wp, b2)


def kernel(x, table, W, b):
    batch, enc = x.shape
    vocab, emb = table.shape
    out_dim = W.shape[0]

    xt = x.T.astype(jnp.int32)
    idx = jnp.where(xt < vocab // 2, 2 * xt, 2 * xt - (vocab - 1))

    table_p = _tc_pad_table(table)
    gathered = _make_gather(batch, enc, vocab)(
        table_p.reshape(vocab, _EMBP), idx
    )

    Wp = jnp.pad(
        W.reshape(out_dim, enc, emb), ((0, 0), (0, 0), (0, _EMBP - emb))
    ).reshape(out_dim, enc * _EMBP)

    return _tc_matmul(gathered, Wp, b.reshape(1, out_dim), batch)


# speedup vs baseline: 2.2470x; 1.1601x over previous
import functools

import jax
import jax.numpy as jnp
from jax import lax
from jax.experimental import pallas as pl
from jax.experimental.pallas import tpu as pltpu
from jax.experimental.pallas import tpu_sc as plsc

_GROUP = 128
_JS_PER_CHUNK = 10
_EMBP = 64


@functools.lru_cache(maxsize=None)
def _make_gather(batch: int, enc: int, vocab: int):
    info = plsc.get_sparse_core_info()
    nw = info.num_cores * info.num_subcores
    assert batch % (nw * _GROUP) == 0 and enc % _JS_PER_CHUNK == 0
    chunks = enc // _JS_PER_CHUNK
    chunk_rows = _GROUP * _JS_PER_CHUNK

    mesh = plsc.VectorSubcoreMesh(core_axis_name="c", subcore_axis_name="s")

    @functools.partial(
        pl.kernel,
        mesh=mesh,
        out_type=jax.ShapeDtypeStruct((batch * enc // 2, 2 * _EMBP), jnp.float32),
        scratch_types=[
            pltpu.VMEM((_JS_PER_CHUNK, _GROUP), jnp.int32),
            pltpu.VMEM((chunk_rows, _EMBP), jnp.float32),
            pltpu.SemaphoreType.DMA,
            pltpu.SemaphoreType.DMA,
        ],
        compiler_params=pltpu.CompilerParams(use_tc_tiling_on_sc=False),
    )
    def gather_k(table_hbm, idx_hbm, out_hbm, idx_v, rows_v, gsem, osem):
        cid = lax.axis_index("c")
        sid = lax.axis_index("s")
        wid = sid * info.num_cores + cid
        b0 = wid * _GROUP

        def chunk_body(c, carry):
            j0 = c * _JS_PER_CHUNK
            pltpu.sync_copy(
                idx_hbm.at[pl.ds(j0, _JS_PER_CHUNK), pl.ds(b0, _GROUP)], idx_v
            )
            handles = []
            for j in range(_JS_PER_CHUNK):
                handles.append(
                    pltpu.async_copy(
                        table_hbm.at[idx_v.at[j]],
                        rows_v.at[pl.ds(j * _GROUP, _GROUP)],
                        gsem,
                    )
                )
            for h in handles:
                h.wait()
            writes = []
            for j in range(_JS_PER_CHUNK):
                p = (j0 + j) // 2
                h = (j0 + j) % 2
                writes.append(
                    pltpu.make_async_copy(
                        rows_v.at[pl.ds(j * _GROUP, _GROUP)],
                        out_hbm.at[
                            pl.ds(batch * p + b0, _GROUP),
                            pl.ds(h * _EMBP, _EMBP),
                        ],
                        osem,
                    )
                )
            for wcp in writes:
                wcp.start()
            for wcp in writes:
                wcp.wait()
            return carry

        lax.fori_loop(0, chunks, chunk_body, 0)

    return gather_k


def _pad_kernel(a1_ref, a2_ref, o_ref):
    bv = a1_ref.shape[0]
    emb = a1_ref.shape[1]
    z = jnp.zeros((bv, _EMBP - emb), jnp.float32)
    o_ref[...] = jnp.concatenate([a1_ref[...], z, a2_ref[...], z], axis=1)


def _tc_pad_table(table):
    vocab, emb = table.shape
    bv = 2000
    half_blocks = (vocab // 2) // bv
    return pl.pallas_call(
        _pad_kernel,
        grid=(half_blocks,),
        in_specs=[
            pl.BlockSpec((bv, emb), lambda c: (c, 0)),
            pl.BlockSpec((bv, emb), lambda c, hb=half_blocks: (c + hb, 0)),
        ],
        out_specs=pl.BlockSpec((bv, 2 * _EMBP), lambda c: (c, 0)),
        out_shape=jax.ShapeDtypeStruct((vocab // 2, 2 * _EMBP), jnp.float32),
    )(table, table)


def _matmul_kernel(a_ref, w_ref, b_ref, o_ref):
    p = pl.program_id(1)
    acc = lax.dot_general(
        a_ref[...].astype(jnp.bfloat16), w_ref[...].astype(jnp.bfloat16),
        dimension_numbers=(((1,), (1,)), ((), ())),
        preferred_element_type=jnp.float32,
    )

    @pl.when(p == 0)
    def _():
        o_ref[...] = jnp.broadcast_to(b_ref[...], o_ref.shape)

    o_ref[...] += acc


def _tc_matmul(gathered, Wp, b2, batch):
    out_dim = Wp.shape[0]
    npairs = Wp.shape[1] // (2 * _EMBP)
    bm = 2048
    nblocks = batch // bm
    return pl.pallas_call(
        _matmul_kernel,
        grid=(nblocks, npairs),
        in_specs=[
            pl.BlockSpec((bm, 2 * _EMBP), lambda i, p: (p * nblocks + i, 0)),
            pl.BlockSpec((out_dim, 2 * _EMBP), lambda i, p: (0, p)),
            pl.BlockSpec((1, out_dim), lambda i, p: (0, 0)),
        ],
        out_specs=pl.BlockSpec((bm, out_dim), lambda i, p: (i, 0)),
        out_shape=jax.ShapeDtypeStruct((batch, out_dim), jnp.float32),
    )(gathered, Wp, b2)


def kernel(x, table, W, b):
    batch, enc = x.shape
    vocab, emb = table.shape
    out_dim = W.shape[0]

    xt = x.T.astype(jnp.int32)
    idx = jnp.where(xt < vocab // 2, 2 * xt, 2 * xt - (vocab - 1))

    table_p = _tc_pad_table(table)
    gathered = _make_gather(batch, enc, vocab)(
        table_p.reshape(vocab, _EMBP), idx
    )

    Wp = jnp.pad(
        W.reshape(out_dim, enc, emb), ((0, 0), (0, 0), (0, _EMBP - emb))
    ).reshape(out_dim, enc * _EMBP)

    return _tc_matmul(gathered, Wp, b.reshape(1, out_dim), batch)


# speedup vs baseline: 2.4549x; 1.0926x over previous
import functools

import jax
import jax.numpy as jnp
from jax import lax
from jax.experimental import pallas as pl
from jax.experimental.pallas import tpu as pltpu
from jax.experimental.pallas import tpu_sc as plsc

_GROUP = 128
_JS_PER_CHUNK = 10
_EMBP = 64


@functools.lru_cache(maxsize=None)
def _make_gather(batch: int, enc: int, vocab: int):
    info = plsc.get_sparse_core_info()
    nw = info.num_cores * info.num_subcores
    assert batch % (nw * _GROUP) == 0 and enc % _JS_PER_CHUNK == 0
    chunks = enc // _JS_PER_CHUNK
    chunk_rows = _GROUP * _JS_PER_CHUNK

    mesh = plsc.VectorSubcoreMesh(core_axis_name="c", subcore_axis_name="s")

    @functools.partial(
        pl.kernel,
        mesh=mesh,
        out_type=jax.ShapeDtypeStruct((batch * enc // 2, 2 * _EMBP), jnp.float32),
        scratch_types=[
            pltpu.VMEM((_JS_PER_CHUNK, _GROUP), jnp.int32),
            pltpu.VMEM((chunk_rows, _EMBP), jnp.float32),
            pltpu.SemaphoreType.DMA,
            pltpu.SemaphoreType.DMA,
        ],
        compiler_params=pltpu.CompilerParams(use_tc_tiling_on_sc=False),
    )
    def gather_k(table_hbm, idx_hbm, out_hbm, idx_v, rows_v, gsem, osem):
        cid = lax.axis_index("c")
        sid = lax.axis_index("s")
        wid = sid * info.num_cores + cid
        b0 = wid * _GROUP

        def chunk_body(c, carry):
            j0 = c * _JS_PER_CHUNK
            pltpu.sync_copy(
                idx_hbm.at[pl.ds(j0, _JS_PER_CHUNK), pl.ds(b0, _GROUP)], idx_v
            )
            handles = []
            for j in range(_JS_PER_CHUNK):
                handles.append(
                    pltpu.async_copy(
                        table_hbm.at[idx_v.at[j]],
                        rows_v.at[pl.ds(j * _GROUP, _GROUP)],
                        gsem,
                    )
                )
            for h in handles:
                h.wait()
            writes = []
            for j in range(_JS_PER_CHUNK):
                p = (j0 + j) // 2
                h = (j0 + j) % 2
                writes.append(
                    pltpu.make_async_copy(
                        rows_v.at[pl.ds(j * _GROUP, _GROUP)],
                        out_hbm.at[
                            pl.ds(batch * p + b0, _GROUP),
                            pl.ds(h * _EMBP, _EMBP),
                        ],
                        osem,
                    )
                )
            for wcp in writes:
                wcp.start()
            for wcp in writes:
                wcp.wait()
            return carry

        lax.fori_loop(0, chunks, chunk_body, 0)

    return gather_k


def _pad_kernel(a1_ref, a2_ref, o_ref):
    bv = a1_ref.shape[0]
    emb = a1_ref.shape[1]
    z = jnp.zeros((bv, _EMBP - emb), jnp.float32)
    o_ref[...] = jnp.concatenate([a1_ref[...], z, a2_ref[...], z], axis=1)


def _tc_pad_table(table):
    vocab, emb = table.shape
    bv = 2000
    half_blocks = (vocab // 2) // bv
    return pl.pallas_call(
        _pad_kernel,
        grid=(half_blocks,),
        in_specs=[
            pl.BlockSpec((bv, emb), lambda c: (c, 0)),
            pl.BlockSpec((bv, emb), lambda c, hb=half_blocks: (c + hb, 0)),
        ],
        out_specs=pl.BlockSpec((bv, 2 * _EMBP), lambda c: (c, 0)),
        out_shape=jax.ShapeDtypeStruct((vocab // 2, 2 * _EMBP), jnp.float32),
    )(table, table)


def _matmul_kernel(a_ref, w_ref, b_ref, o_ref):
    p = pl.program_id(1)
    acc = lax.dot_general(
        a_ref[...].astype(jnp.bfloat16), w_ref[...].astype(jnp.bfloat16),
        dimension_numbers=(((1,), (1,)), ((), ())),
        preferred_element_type=jnp.float32,
    )

    @pl.when(p == 0)
    def _():
        o_ref[...] = jnp.broadcast_to(b_ref[...], o_ref.shape)

    o_ref[...] += acc


def _tc_matmul(gathered, Wp, b2, batch):
    out_dim = Wp.shape[0]
    npairs = Wp.shape[1] // (2 * _EMBP)
    bm = 4096
    nblocks = batch // bm
    return pl.pallas_call(
        _matmul_kernel,
        grid=(nblocks, npairs),
        in_specs=[
            pl.BlockSpec((bm, 2 * _EMBP), lambda i, p: (p * nblocks + i, 0)),
            pl.BlockSpec((out_dim, 2 * _EMBP), lambda i, p: (0, p)),
            pl.BlockSpec((1, out_dim), lambda i, p: (0, 0)),
        ],
        out_specs=pl.BlockSpec((bm, out_dim), lambda i, p: (i, 0)),
        out_shape=jax.ShapeDtypeStruct((batch, out_dim), jnp.float32),
    )(gathered, Wp, b2)


def kernel(x, table, W, b):
    batch, enc = x.shape
    vocab, emb = table.shape
    out_dim = W.shape[0]

    xt = x.T.astype(jnp.int32)
    idx = jnp.where(xt < vocab // 2, 2 * xt, 2 * xt - (vocab - 1))

    table_p = _tc_pad_table(table)
    gathered = _make_gather(batch, enc, vocab)(
        table_p.reshape(vocab, _EMBP), idx
    )

    Wp = jnp.pad(
        W.reshape(out_dim, enc, emb), ((0, 0), (0, 0), (0, _EMBP - emb))
    ).reshape(out_dim, enc * _EMBP)

    return _tc_matmul(gathered, Wp, b.reshape(1, out_dim), batch)
